# Initial kernel scaffold; baseline (speedup 1.0000x reference)
import jax
import jax.numpy as jnp
from jax.experimental import pallas as pl


def kernel(char_input, lengths, embedding):
    raise NotImplementedError("write your pallas kernel here")


# speedup vs baseline: 10.0863x; 10.0863x over previous
import functools

import jax
import jax.numpy as jnp
from jax import lax
from jax.experimental import pallas as pl
from jax.experimental.pallas import tpu as pltpu
from jax.experimental.pallas import tpu_sc as plsc

N_CHARS = 512
EMB_DIM = 128
T = 16
L = 16


def _build(n_words):
    info = plsc.get_sparse_core_info()
    nc, ns = info.num_cores, info.num_subcores
    nw = nc * ns
    W = n_words // nw
    CHUNK = 256
    n_chunks = W // CHUNK
    mesh = plsc.VectorSubcoreMesh(core_axis_name="c", subcore_axis_name="s")

    @functools.partial(
        pl.kernel,
        mesh=mesh,
        compiler_params=pltpu.CompilerParams(needs_layout_passes=False, use_tc_tiling_on_sc=False),
        out_type=jax.ShapeDtypeStruct((n_words, EMB_DIM), jnp.float32),
        scratch_types=[
            pltpu.VMEM((N_CHARS, EMB_DIM), jnp.float32),
            pltpu.VMEM((W, T), jnp.int32),
            pltpu.VMEM((W,), jnp.int32),
            pltpu.VMEM((CHUNK, EMB_DIM), jnp.float32),
        ],
    )
    def k(ci_hbm, ln_hbm, emb_hbm, out_hbm, table_v, chars_v, lens_v, outc_v):
        wid = lax.axis_index("s") * nc + lax.axis_index("c")
        base = wid * W
        pltpu.sync_copy(emb_hbm, table_v)
        pltpu.sync_copy(ci_hbm.at[pl.ds(base, W)], chars_v)
        pltpu.sync_copy(ln_hbm.at[pl.ds(base, W)], lens_v)

        iota = lax.iota(jnp.int32, L)
        cols = [iota + k0 * L for k0 in range(EMB_DIM // L)]

        for ch in range(n_chunks):
            def word_body(w, carry):
                wg = ch * CHUNK + w
                wf = jnp.full((L,), wg, jnp.int32)
                len_spl = plsc.load_gather(lens_v, [wf])
                cv = chars_v[wg, :]
                cc = jnp.where(iota < len_spl, cv, 0)
                inv = 1.0 / jnp.maximum(len_spl, 1).astype(jnp.float32)
                accs = [jnp.zeros((L,), jnp.float32) for _ in range(EMB_DIM // L)]
                for t in range(T):
                    spl = jnp.full((L,), cc[t], jnp.int32)
                    for k0 in range(EMB_DIM // L):
                        accs[k0] = accs[k0] + plsc.load_gather(table_v, [spl, cols[k0]])
                for k0 in range(EMB_DIM // L):
                    outc_v[w, pl.ds(k0 * L, L)] = accs[k0] * inv
                return carry

            lax.fori_loop(0, CHUNK, word_body, 0)
            pltpu.sync_copy(outc_v, out_hbm.at[pl.ds(base + ch * CHUNK, CHUNK)])

    return k


def kernel(char_input, lengths, embedding):
    b, s, t = char_input.shape
    n = b * s
    ci = char_input.reshape(n, t)
    ln = lengths.reshape(n)
    out = _build(n)(ci, ln, embedding)
    return out.reshape(b, s, EMB_DIM), ln


# speedup vs baseline: 11.7972x; 1.1696x over previous
import functools

import jax
import jax.numpy as jnp
from jax import lax
from jax.experimental import pallas as pl
from jax.experimental.pallas import tpu as pltpu
from jax.experimental.pallas import tpu_sc as plsc

N_CHARS = 512
EMB_DIM = 128
T = 16
L = 16


def _build(n_words):
    info = plsc.get_sparse_core_info()
    nc, ns = info.num_cores, info.num_subcores
    nw = nc * ns
    W = n_words // nw
    CHUNK = 256
    n_chunks = W // CHUNK
    mesh = plsc.VectorSubcoreMesh(core_axis_name="c", subcore_axis_name="s")

    @functools.partial(
        pl.kernel,
        mesh=mesh,
        compiler_params=pltpu.CompilerParams(needs_layout_passes=False, use_tc_tiling_on_sc=False),
        out_type=jax.ShapeDtypeStruct((n_words, EMB_DIM), jnp.float32),
        scratch_types=[
            pltpu.VMEM((N_CHARS, EMB_DIM), jnp.float32),
            pltpu.VMEM((W, T), jnp.int32),
            pltpu.VMEM((W,), jnp.int32),
            pltpu.VMEM((CHUNK, EMB_DIM), jnp.float32),
        ],
    )
    def k(ci_hbm, ln_hbm, emb_hbm, out_hbm, table_v, chars_v, lens_v, outc_v):
        wid = lax.axis_index("s") * nc + lax.axis_index("c")
        base = wid * W
        pltpu.sync_copy(emb_hbm, table_v)
        pltpu.sync_copy(ci_hbm.at[pl.ds(base, W)], chars_v)
        pltpu.sync_copy(ln_hbm.at[pl.ds(base, W)], lens_v)

        iota = lax.iota(jnp.int32, L)
        cols = [iota + k0 * L for k0 in range(EMB_DIM // L)]

        for ch in range(n_chunks):
            def word_body(w, carry):
                wg = ch * CHUNK + w
                wf = jnp.full((L,), wg, jnp.int32)
                len_spl = plsc.load_gather(lens_v, [wf])
                len_s = jnp.max(len_spl, axis=0)
                inv = 1.0 / jnp.maximum(len_spl, 1).astype(jnp.float32)

                def t_body(t, accs):
                    tf = jnp.full((L,), t, jnp.int32)
                    spl = plsc.load_gather(chars_v, [wf, tf])
                    return tuple(
                        a + plsc.load_gather(table_v, [spl, c])
                        for a, c in zip(accs, cols)
                    )

                zero = jnp.zeros((L,), jnp.float32)
                accs = lax.fori_loop(
                    0, len_s, t_body,
                    tuple(zero for _ in range(EMB_DIM // L)))
                for k0 in range(EMB_DIM // L):
                    outc_v[w, pl.ds(k0 * L, L)] = accs[k0] * inv
                return carry

            lax.fori_loop(0, CHUNK, word_body, 0)
            pltpu.sync_copy(outc_v, out_hbm.at[pl.ds(base + ch * CHUNK, CHUNK)])

    return k


def kernel(char_input, lengths, embedding):
    b, s, t = char_input.shape
    n = b * s
    ci = char_input.reshape(n, t)
    ln = lengths.reshape(n)
    out = _build(n)(ci, ln, embedding)
    return out.reshape(b, s, EMB_DIM), ln


# speedup vs baseline: 12.1958x; 1.0338x over previous
import functools

import jax
import jax.numpy as jnp
from jax import lax
from jax.experimental import pallas as pl
from jax.experimental.pallas import tpu as pltpu
from jax.experimental.pallas import tpu_sc as plsc

N_CHARS = 512
EMB_DIM = 128
T = 16
L = 16
HALF = EMB_DIM // 2


def _build(n_words):
    info = plsc.get_sparse_core_info()
    nc, ns = info.num_cores, info.num_subcores
    nw = nc * ns
    W = n_words // nw
    CHUNK = 256
    n_chunks = W // CHUNK
    mesh = plsc.VectorSubcoreMesh(core_axis_name="c", subcore_axis_name="s")

    @functools.partial(
        pl.kernel,
        mesh=mesh,
        compiler_params=pltpu.CompilerParams(
            needs_layout_passes=False, use_tc_tiling_on_sc=False),
        out_type=jax.ShapeDtypeStruct((n_words, EMB_DIM), jnp.float32),
        scratch_types=[
            pltpu.VMEM((N_CHARS, HALF), jnp.int32),
            pltpu.VMEM((W, T), jnp.int32),
            pltpu.VMEM((W,), jnp.int32),
            pltpu.VMEM((CHUNK, EMB_DIM), jnp.float32),
        ],
    )
    def k(ci_hbm, ln_hbm, emb_hbm, out_hbm, table_v, chars_v, lens_v, outc_v):
        wid = lax.axis_index("s") * nc + lax.axis_index("c")
        base = wid * W
        pltpu.sync_copy(emb_hbm, table_v)
        pltpu.sync_copy(ci_hbm.at[pl.ds(base, W)], chars_v)
        pltpu.sync_copy(ln_hbm.at[pl.ds(base, W)], lens_v)

        iota = lax.iota(jnp.int32, L)
        cols = [iota + k0 * L for k0 in range(HALF // L)]

        for ch in range(n_chunks):
            def word_body(w, carry):
                wg = ch * CHUNK + w
                wf = jnp.full((L,), wg, jnp.int32)
                len_spl = plsc.load_gather(lens_v, [wf])
                len_s = jnp.max(len_spl, axis=0)
                inv = 1.0 / jnp.maximum(len_spl, 1).astype(jnp.float32)

                def t_body(t, accs):
                    tf = jnp.full((L,), t, jnp.int32)
                    spl = plsc.load_gather(chars_v, [wf, tf])
                    new = list(accs)
                    for k0 in range(HALF // L):
                        g = plsc.load_gather(table_v, [spl, cols[k0]])
                        lo, hi = plsc.unpack(
                            plsc.bitcast(g, jnp.bfloat16),
                            format=plsc.PackFormat.INTERLEAVED)
                        new[k0] = new[k0] + lo
                        new[k0 + 4] = new[k0 + 4] + hi
                    return tuple(new)

                zero = jnp.zeros((L,), jnp.float32)
                accs = lax.fori_loop(
                    0, len_s, t_body,
                    tuple(zero for _ in range(EMB_DIM // L)))
                for k0 in range(EMB_DIM // L):
                    outc_v[w, pl.ds(k0 * L, L)] = accs[k0] * inv
                return carry

            lax.fori_loop(0, CHUNK, word_body, 0)
            pltpu.sync_copy(outc_v, out_hbm.at[pl.ds(base + ch * CHUNK, CHUNK)])

    return k


def kernel(char_input, lengths, embedding):
    b, s, t = char_input.shape
    n = b * s
    ci = char_input.reshape(n, t)
    ln = lengths.reshape(n)
    emb_bf = embedding.astype(jnp.bfloat16)
    packed = jax.lax.bitcast_convert_type(
        jnp.stack([emb_bf[:, :HALF], emb_bf[:, HALF:]], axis=-1), jnp.int32)
    out = _build(n)(ci, ln, packed)
    return out.reshape(b, s, EMB_DIM), ln


# speedup vs baseline: 12.3001x; 1.0086x over previous
import functools

import jax
import jax.numpy as jnp
from jax import lax
from jax.experimental import pallas as pl
from jax.experimental.pallas import tpu as pltpu
from jax.experimental.pallas import tpu_sc as plsc

N_CHARS = 512
EMB_DIM = 128
T = 16
L = 16
HALF = EMB_DIM // 2


def _build(n_words):
    info = plsc.get_sparse_core_info()
    nc, ns = info.num_cores, info.num_subcores
    nw = nc * ns
    W = n_words // nw
    CHUNK = 256
    n_chunks = W // CHUNK
    mesh = plsc.VectorSubcoreMesh(core_axis_name="c", subcore_axis_name="s")

    @functools.partial(
        pl.kernel,
        mesh=mesh,
        compiler_params=pltpu.CompilerParams(
            needs_layout_passes=False, use_tc_tiling_on_sc=False),
        out_type=jax.ShapeDtypeStruct((n_words, EMB_DIM), jnp.float32),
        scratch_types=[
            pltpu.VMEM((N_CHARS, HALF), jnp.int32),
            pltpu.VMEM((W, T), jnp.int32),
            pltpu.VMEM((W,), jnp.int32),
            pltpu.VMEM((CHUNK, EMB_DIM), jnp.float32),
        ],
    )
    def k(ci_hbm, ln_hbm, emb_hbm, out_hbm, table_v, chars_v, lens_v, outc_v):
        wid = lax.axis_index("s") * nc + lax.axis_index("c")
        base = wid * W
        pltpu.sync_copy(emb_hbm, table_v)
        pltpu.sync_copy(ci_hbm.at[pl.ds(base, W)], chars_v)
        pltpu.sync_copy(ln_hbm.at[pl.ds(base, W)], lens_v)

        iota = lax.iota(jnp.int32, L)
        cols = [iota + k0 * L for k0 in range(HALF // L)]

        def clean_body(w, carry):
            wf = jnp.full((L,), w, jnp.int32)
            len_spl = plsc.load_gather(lens_v, [wf])
            cv = chars_v[w, :]
            chars_v[w, :] = jnp.where(iota < len_spl, cv, 0)
            return carry

        lax.fori_loop(0, W, clean_body, 0)

        HP = CHUNK // 2
        for ch in range(n_chunks):
            def pair_body(p, carry):
                wa = ch * CHUNK + p
                wfa = jnp.full((L,), wa, jnp.int32)
                wfb = wfa + HP
                la = plsc.load_gather(lens_v, [wfa])
                lb = plsc.load_gather(lens_v, [wfb])
                len_s = jnp.max(jnp.maximum(la, lb), axis=0)
                inva = 1.0 / jnp.maximum(la, 1).astype(jnp.float32)
                invb = 1.0 / jnp.maximum(lb, 1).astype(jnp.float32)

                def t_body(t, accs):
                    tf = jnp.full((L,), t, jnp.int32)
                    spl_a = plsc.load_gather(chars_v, [wfa, tf])
                    spl_b = plsc.load_gather(chars_v, [wfb, tf])
                    new = list(accs)
                    for k0 in range(HALF // L):
                        ga = plsc.load_gather(table_v, [spl_a, cols[k0]])
                        gb = plsc.load_gather(table_v, [spl_b, cols[k0]])
                        alo, ahi = plsc.unpack(
                            plsc.bitcast(ga, jnp.bfloat16),
                            format=plsc.PackFormat.INTERLEAVED)
                        blo, bhi = plsc.unpack(
                            plsc.bitcast(gb, jnp.bfloat16),
                            format=plsc.PackFormat.INTERLEAVED)
                        new[k0] = new[k0] + alo
                        new[k0 + 4] = new[k0 + 4] + ahi
                        new[k0 + 8] = new[k0 + 8] + blo
                        new[k0 + 12] = new[k0 + 12] + bhi
                    return tuple(new)

                zero = jnp.zeros((L,), jnp.float32)
                accs = lax.fori_loop(
                    0, len_s, t_body, tuple(zero for _ in range(16)))
                for k0 in range(EMB_DIM // L):
                    outc_v[p, pl.ds(k0 * L, L)] = accs[k0] * inva
                    outc_v[p + HP, pl.ds(k0 * L, L)] = accs[k0 + 8] * invb
                return carry

            lax.fori_loop(0, HP, pair_body, 0)
            pltpu.sync_copy(outc_v, out_hbm.at[pl.ds(base + ch * CHUNK, CHUNK)])

    return k


def kernel(char_input, lengths, embedding):
    b, s, t = char_input.shape
    n = b * s
    ci = char_input.reshape(n, t)
    ln = lengths.reshape(n)
    emb_bf = embedding.astype(jnp.bfloat16)
    packed = jax.lax.bitcast_convert_type(
        jnp.stack([emb_bf[:, :HALF], emb_bf[:, HALF:]], axis=-1), jnp.int32)
    out = _build(n)(ci, ln, packed)
    return out.reshape(b, s, EMB_DIM), ln


# speedup vs baseline: 14.3558x; 1.1671x over previous
import functools

import jax
import jax.numpy as jnp
from jax import lax
from jax.experimental import pallas as pl
from jax.experimental.pallas import tpu as pltpu
from jax.experimental.pallas import tpu_sc as plsc

N_CHARS = 512
EMB_DIM = 128
T = 16
L = 16
HALF = EMB_DIM // 2


def _build(n_words):
    info = plsc.get_sparse_core_info()
    nc, ns = info.num_cores, info.num_subcores
    nw = nc * ns
    W = n_words // nw
    n_groups = W // L
    mesh = plsc.VectorSubcoreMesh(core_axis_name="c", subcore_axis_name="s")

    @functools.partial(
        pl.kernel,
        mesh=mesh,
        compiler_params=pltpu.CompilerParams(
            needs_layout_passes=False, use_tc_tiling_on_sc=False),
        out_type=jax.ShapeDtypeStruct((n_words, EMB_DIM), jnp.float32),
        scratch_types=[
            pltpu.VMEM((N_CHARS * HALF,), jnp.int32),
            pltpu.VMEM((W, T), jnp.int32),
            pltpu.VMEM((W,), jnp.int32),
            pltpu.VMEM((W + L,), jnp.int32),
            pltpu.VMEM((W, EMB_DIM), jnp.float32),
        ],
    )
    def k(ci_hbm, ln_hbm, emb_hbm, out_hbm, table_v, chars_v, lens_v,
          sidx_v, outf_v):
        wid = lax.axis_index("s") * nc + lax.axis_index("c")
        base = wid * W
        pltpu.sync_copy(emb_hbm, table_v)
        pltpu.sync_copy(ci_hbm.at[pl.ds(base, W)], chars_v)
        pltpu.sync_copy(ln_hbm.at[pl.ds(base, W)], lens_v)

        iota = lax.iota(jnp.int32, L)
        cols = [iota + k0 * L for k0 in range(HALF // L)]
        ocols = [iota + k0 * L for k0 in range(EMB_DIM // L)]

        def count_body(g, cnts):
            lv = lens_v[pl.ds(g * L, L)]
            new = []
            for l0 in range(T + 1):
                m = lv == l0
                c = plsc.all_reduce_population_count(m)
                new.append(cnts[l0] + c[0])
            return tuple(new)

        counts = lax.fori_loop(
            0, n_groups, count_body,
            tuple(jnp.int32(0) for _ in range(T + 1)))

        starts = []
        acc = jnp.int32(0)
        for l0 in range(T + 1):
            starts.append(acc)
            acc = acc + counts[l0]

        def fill_body(g, curs):
            lv = lens_v[pl.ds(g * L, L)]
            widx = iota + g * L
            new = []
            for l0 in range(T + 1):
                m = lv == l0
                plsc.store_compressed(
                    sidx_v.at[pl.ds(curs[l0], L)], widx, mask=m)
                c = plsc.all_reduce_population_count(m)
                new.append(curs[l0] + c[0])
            return tuple(new)

        lax.fori_loop(0, n_groups, fill_body, tuple(starts))

        zero = jnp.zeros((L,), jnp.float32)

        def zero_body(i, carry):
            posf = jnp.full((L,), starts[0] + i, jnp.int32)
            wf = plsc.load_gather(sidx_v, [posf])
            for k0 in range(EMB_DIM // L):
                plsc.store_scatter(outf_v, [wf, ocols[k0]], zero)
            return carry

        lax.fori_loop(0, counts[0], zero_body, 0)

        for l0 in range(1, T + 1):
            inv = jnp.float32(1.0 / l0)

            def len_body(i, carry, l0=l0, inv=inv):
                posf = jnp.full((L,), starts[l0] + i, jnp.int32)
                wf = plsc.load_gather(sidx_v, [posf])
                cv = plsc.load_gather(chars_v, [wf, iota])
                sh = cv << 6
                accs = [zero] * (EMB_DIM // L)
                for t in range(l0):
                    spl = jnp.full((L,), sh[t], jnp.int32)
                    for k0 in range(HALF // L):
                        g = plsc.load_gather(table_v, [spl + cols[k0]])
                        lo, hi = plsc.unpack(
                            plsc.bitcast(g, jnp.bfloat16),
                            format=plsc.PackFormat.INTERLEAVED)
                        accs[k0] = accs[k0] + lo
                        accs[k0 + 4] = accs[k0 + 4] + hi
                for k0 in range(EMB_DIM // L):
                    plsc.store_scatter(
                        outf_v, [wf, ocols[k0]], accs[k0] * inv)
                return carry

            lax.fori_loop(0, counts[l0], len_body, 0)

        pltpu.sync_copy(outf_v, out_hbm.at[pl.ds(base, W)])

    return k


def kernel(char_input, lengths, embedding):
    b, s, t = char_input.shape
    n = b * s
    ci = char_input.reshape(n, t)
    ln = lengths.reshape(n)
    emb_bf = embedding.astype(jnp.bfloat16)
    packed = jax.lax.bitcast_convert_type(
        jnp.stack([emb_bf[:, :HALF], emb_bf[:, HALF:]], axis=-1), jnp.int32)
    out = _build(n)(ci, ln, packed.reshape(-1))
    return out.reshape(b, s, EMB_DIM), ln
